# Initial kernel scaffold; baseline (speedup 1.0000x reference)
import jax
import jax.numpy as jnp
from jax.experimental import pallas as pl


def kernel(x, mat):
    raise NotImplementedError("write your pallas kernel here")


# speedup vs baseline: 47.1846x; 47.1846x over previous
import jax
import jax.numpy as jnp
from jax import lax
from jax.experimental import pallas as pl
from jax.experimental.pallas import tpu as pltpu
from jax.experimental.pallas import tpu_sc as plsc

RES = 512
FEAT = 32
NC = 2
NS = 16
NW = NC * NS
L = 16
CH = 128


def _tri_body(t0, t1, t2, x0, x1, x2, out,
              c0, c1, c2, idxs, wts, rows, outv, sem):
    B = out.shape[0]
    pw = B // NW
    nch = pw // CH
    wid = lax.axis_index("s") * NC + lax.axis_index("c")
    base0 = wid * pw
    tabs = (t0, t1, t2)

    def compute_chunk(g):
        base = base0 + g * CH
        pltpu.sync_copy(x0.at[pl.ds(base, CH)], c0)
        pltpu.sync_copy(x1.at[pl.ds(base, CH)], c1)
        pltpu.sync_copy(x2.at[pl.ds(base, CH)], c2)

        def grp(g2, carry):
            s = g2 * L
            u0 = c0[pl.ds(s, L)]
            u1 = c1[pl.ds(s, L)]
            u2 = c2[pl.ds(s, L)]
            for p, (ua, ub) in enumerate(((u0, u1), (u1, u2), (u2, u0))):
                uu = ua * (RES - 1.0)
                vv = ub * (RES - 1.0)
                iu = jnp.clip(uu.astype(jnp.int32), 0, RES - 1)
                jv = jnp.clip(vv.astype(jnp.int32), 0, RES - 1)
                wi = uu - iu.astype(jnp.float32)
                wj = vv - jv.astype(jnp.float32)
                di = jnp.minimum(iu + 1, RES - 1) - iu
                dj = jnp.minimum(jv + 1, RES - 1) - jv
                b00 = iu * RES + jv
                r = 4 * p
                idxs[r + 0, pl.ds(s, L)] = b00
                idxs[r + 1, pl.ds(s, L)] = b00 + dj
                idxs[r + 2, pl.ds(s, L)] = b00 + di * RES
                idxs[r + 3, pl.ds(s, L)] = b00 + di * RES + dj
                oi = 1.0 - wi
                oj = 1.0 - wj
                wts[r + 0, pl.ds(s, L)] = oi * oj
                wts[r + 1, pl.ds(s, L)] = oi * wj
                wts[r + 2, pl.ds(s, L)] = wi * oj
                wts[r + 3, pl.ds(s, L)] = wi * wj
            return carry

        lax.fori_loop(0, CH // L, grp, 0)

    def issue_gathers():
        return [pltpu.async_copy(tabs[r // 4].at[idxs.at[r]], rows.at[r], sem)
                for r in range(12)]

    def blend(g):
        def grp(g2, carry):
            s = g2 * L
            for p in range(3):
                r = 4 * p
                w00v = wts[r + 0, pl.ds(s, L)]
                w01v = wts[r + 1, pl.ds(s, L)]
                w10v = wts[r + 2, pl.ds(s, L)]
                w11v = wts[r + 3, pl.ds(s, L)]
                for i in range(L):
                    c = s + i
                    w00 = w00v[i]
                    w01 = w01v[i]
                    w10 = w10v[i]
                    w11 = w11v[i]
                    for h in range(FEAT // L):
                        sl = pl.ds(h * L, L)
                        acc = (rows[r + 0, c, sl] * w00
                               + rows[r + 1, c, sl] * w01
                               + rows[r + 2, c, sl] * w10
                               + rows[r + 3, c, sl] * w11)
                        outv[c, pl.ds(p * FEAT + h * L, L)] = acc
            return carry

        lax.fori_loop(0, CH // L, grp, 0)
        base = base0 + g * CH
        pltpu.sync_copy(outv, out.at[pl.ds(base, CH)])

    def chunk(g, carry):
        compute_chunk(g)
        for h in issue_gathers():
            h.wait()
        blend(g)
        return carry

    lax.fori_loop(0, nch, chunk, 0)


def kernel(x, mat):
    B = x.shape[0]
    assert B % (NW * CH) == 0
    tab = jnp.transpose(mat, (0, 2, 3, 1)).reshape(3, RES * RES, FEAT)
    mesh = plsc.VectorSubcoreMesh(core_axis_name="c", subcore_axis_name="s",
                                  num_cores=NC, num_subcores=NS)
    f = pl.kernel(
        _tri_body,
        out_type=jax.ShapeDtypeStruct((B, 3 * FEAT), jnp.float32),
        mesh=mesh,
        compiler_params=pltpu.CompilerParams(use_tc_tiling_on_sc=False),
        scratch_types=[
            pltpu.VMEM((CH,), jnp.float32),
            pltpu.VMEM((CH,), jnp.float32),
            pltpu.VMEM((CH,), jnp.float32),
            pltpu.VMEM((12, CH), jnp.int32),
            pltpu.VMEM((12, CH), jnp.float32),
            pltpu.VMEM((12, CH, FEAT), jnp.float32),
            pltpu.VMEM((CH, 3 * FEAT), jnp.float32),
            pltpu.SemaphoreType.DMA,
        ],
    )
    return f(tab[0], tab[1], tab[2], x[:, 0], x[:, 1], x[:, 2])


# speedup vs baseline: 56.4279x; 1.1959x over previous
import jax
import jax.numpy as jnp
from jax import lax
from jax.experimental import pallas as pl
from jax.experimental.pallas import tpu as pltpu
from jax.experimental.pallas import tpu_sc as plsc

RES = 512
FEAT = 32
NC = 2
NS = 16
NW = NC * NS
L = 16
CH = 128


def _tri_body(t0, t1, t2, x0, x1, x2, out,
              c0, c1, c2, idxs, wts, rows, outv, sg0, sg1, osem):
    B = out.shape[0]
    pw = B // NW
    nch = pw // CH
    wid = lax.axis_index("s") * NC + lax.axis_index("c")
    base0 = wid * pw
    tabs = (t0, t1, t2)
    gsems = (sg0, sg1)

    def compute_chunk(g, buf):
        base = base0 + g * CH
        pltpu.sync_copy(x0.at[pl.ds(base, CH)], c0)
        pltpu.sync_copy(x1.at[pl.ds(base, CH)], c1)
        pltpu.sync_copy(x2.at[pl.ds(base, CH)], c2)

        def grp(g2, carry):
            s = g2 * L
            u0 = c0[pl.ds(s, L)]
            u1 = c1[pl.ds(s, L)]
            u2 = c2[pl.ds(s, L)]
            for p, (ua, ub) in enumerate(((u0, u1), (u1, u2), (u2, u0))):
                uu = ua * (RES - 1.0)
                vv = ub * (RES - 1.0)
                iu = jnp.clip(uu.astype(jnp.int32), 0, RES - 1)
                jv = jnp.clip(vv.astype(jnp.int32), 0, RES - 1)
                wi = uu - iu.astype(jnp.float32)
                wj = vv - jv.astype(jnp.float32)
                di = jnp.minimum(iu + 1, RES - 1) - iu
                dj = jnp.minimum(jv + 1, RES - 1) - jv
                b00 = iu * RES + jv
                r = 4 * p
                idxs[buf, r + 0, pl.ds(s, L)] = b00
                idxs[buf, r + 1, pl.ds(s, L)] = b00 + dj
                idxs[buf, r + 2, pl.ds(s, L)] = b00 + di * RES
                idxs[buf, r + 3, pl.ds(s, L)] = b00 + di * RES + dj
                oi = 1.0 - wi
                oj = 1.0 - wj
                wts[buf, r + 0, pl.ds(s, L)] = oi * oj
                wts[buf, r + 1, pl.ds(s, L)] = oi * wj
                wts[buf, r + 2, pl.ds(s, L)] = wi * oj
                wts[buf, r + 3, pl.ds(s, L)] = wi * wj
            return carry

        lax.fori_loop(0, CH // L, grp, 0)

    def gather_descs(buf):
        return [pltpu.make_async_copy(tabs[r // 4].at[idxs.at[buf, r]],
                                      rows.at[buf, r], gsems[buf])
                for r in range(12)]

    def issue(buf):
        for d in gather_descs(buf):
            d.start()

    def drain(buf):
        for d in gather_descs(buf):
            d.wait()

    def out_desc(g):
        base = base0 + g * CH
        return pltpu.make_async_copy(outv, out.at[pl.ds(base, CH)], osem)

    def blend(g, buf):
        @pl.when(g >= 1)
        def _():
            out_desc(g).wait()

        def grp(g2, carry):
            s = g2 * L
            for p in range(3):
                r = 4 * p
                w00v = wts[buf, r + 0, pl.ds(s, L)]
                w01v = wts[buf, r + 1, pl.ds(s, L)]
                w10v = wts[buf, r + 2, pl.ds(s, L)]
                w11v = wts[buf, r + 3, pl.ds(s, L)]
                for i in range(L):
                    c = s + i
                    w00 = w00v[i]
                    w01 = w01v[i]
                    w10 = w10v[i]
                    w11 = w11v[i]
                    for h in range(FEAT // L):
                        sl = pl.ds(h * L, L)
                        acc = (rows[buf, r + 0, c, sl] * w00
                               + rows[buf, r + 1, c, sl] * w01
                               + rows[buf, r + 2, c, sl] * w10
                               + rows[buf, r + 3, c, sl] * w11)
                        outv[c, pl.ds(p * FEAT + h * L, L)] = acc
            return carry

        lax.fori_loop(0, CH // L, grp, 0)
        out_desc(g).start()

    compute_chunk(0, 0)
    issue(0)

    def body(i, carry):
        g0 = 2 * i
        g1 = 2 * i + 1
        g2w = (2 * i + 2) & (nch - 1)
        compute_chunk(g1, 1)
        issue(1)
        drain(0)
        blend(g0, 0)
        compute_chunk(g2w, 0)
        issue(0)
        drain(1)
        blend(g1, 1)
        return carry

    lax.fori_loop(0, nch // 2, body, 0)
    drain(0)
    out_desc(0).wait()


def kernel(x, mat):
    B = x.shape[0]
    assert B % (NW * CH) == 0
    tab = jnp.transpose(mat, (0, 2, 3, 1)).reshape(3, RES * RES, FEAT)
    mesh = plsc.VectorSubcoreMesh(core_axis_name="c", subcore_axis_name="s",
                                  num_cores=NC, num_subcores=NS)
    f = pl.kernel(
        _tri_body,
        out_type=jax.ShapeDtypeStruct((B, 3 * FEAT), jnp.float32),
        mesh=mesh,
        compiler_params=pltpu.CompilerParams(use_tc_tiling_on_sc=False),
        scratch_types=[
            pltpu.VMEM((CH,), jnp.float32),
            pltpu.VMEM((CH,), jnp.float32),
            pltpu.VMEM((CH,), jnp.float32),
            pltpu.VMEM((2, 12, CH), jnp.int32),
            pltpu.VMEM((2, 12, CH), jnp.float32),
            pltpu.VMEM((2, 12, CH, FEAT), jnp.float32),
            pltpu.VMEM((CH, 3 * FEAT), jnp.float32),
            pltpu.SemaphoreType.DMA,
            pltpu.SemaphoreType.DMA,
            pltpu.SemaphoreType.DMA,
        ],
    )
    return f(tab[0], tab[1], tab[2], x[:, 0], x[:, 1], x[:, 2])
